# initial kernel scaffold (unmeasured)
import jax
import jax.numpy as jnp
from jax import lax
from jax.experimental import pallas as pl
from jax.experimental.pallas import tpu as pltpu


def kernel(
    x,
):
    def body(*refs):
        pass

    out_shape = jax.ShapeDtypeStruct(..., jnp.float32)
    return pl.pallas_call(body, out_shape=out_shape)(...)


# baseline (device time: 193845 ns/iter reference)
import jax
import jax.numpy as jnp
from jax import lax
from jax.experimental import pallas as pl
from jax.experimental.pallas import tpu as pltpu

CHUNKS = 4


def kernel(x):
    _, m, n2 = x.shape
    n = n2 // 2
    rows = m // CHUNKS

    def body(x_hbm, out_ref, stage, send_buf,
             copy_sem, send_sems, recv_sems):
        my_x = lax.axis_index("x")
        my_y = lax.axis_index("y")
        my_z = lax.axis_index("z")
        peer = (my_x, my_y, 1 - my_z)

        barrier_sem = pltpu.get_barrier_semaphore()
        pl.semaphore_signal(
            barrier_sem, inc=1, device_id=peer,
            device_id_type=pl.DeviceIdType.MESH,
        )
        pl.semaphore_wait(barrier_sem, 1)

        peer_off = (1 - my_z) * n
        my_off = my_z * n

        def chunk_rdma(c, slot):
            return pltpu.make_async_remote_copy(
                src_ref=send_buf.at[slot],
                dst_ref=out_ref.at[pl.ds(c * rows, rows), :],
                send_sem=send_sems.at[slot],
                recv_sem=recv_sems.at[c],
                device_id=peer,
                device_id_type=pl.DeviceIdType.MESH,
            )

        sends = [None, None]
        for c in range(CHUNKS):
            slot = c % 2
            cp = pltpu.make_async_copy(
                x_hbm.at[0, pl.ds(c * rows, rows), pl.ds(peer_off, n)],
                stage.at[slot], copy_sem,
            )
            cp.start()
            cp.wait()
            if sends[slot] is not None:
                sends[slot].wait_send()
            send_buf[slot, :, :] = stage[slot, :, :].astype(jnp.bfloat16)
            rdma = chunk_rdma(c, slot)
            rdma.start()
            sends[slot] = rdma

        for c in range(CHUNKS):
            slot = c % 2
            cp = pltpu.make_async_copy(
                x_hbm.at[0, pl.ds(c * rows, rows), pl.ds(my_off, n)],
                stage.at[slot], copy_sem,
            )
            cp.start()
            chunk_rdma(c, slot).wait_recv()
            cp.wait()
            out_ref[pl.ds(c * rows, rows), :] = (
                out_ref[pl.ds(c * rows, rows), :]
                + stage[slot, :, :].astype(jnp.bfloat16)
            )

        for s in sends:
            s.wait_send()

    return pl.pallas_call(
        body,
        out_shape=jax.ShapeDtypeStruct((m, n), jnp.bfloat16),
        in_specs=[pl.BlockSpec(memory_space=pl.ANY)],
        out_specs=pl.BlockSpec(memory_space=pltpu.VMEM),
        scratch_shapes=[
            pltpu.VMEM((2, rows, n), jnp.float32),
            pltpu.VMEM((2, rows, n), jnp.bfloat16),
            pltpu.SemaphoreType.DMA,
            pltpu.SemaphoreType.DMA((2,)),
            pltpu.SemaphoreType.DMA((CHUNKS,)),
        ],
        compiler_params=pltpu.CompilerParams(collective_id=0),
    )(x)


# device time: 139606 ns/iter; 1.3885x vs baseline; 1.3885x over previous
import jax
import jax.numpy as jnp
from jax import lax
from jax.experimental import pallas as pl
from jax.experimental.pallas import tpu as pltpu

S = 8
H = S // 2


def kernel(x):
    _, m, n2 = x.shape
    n = n2 // 2
    QR = m // 4
    SR = QR // S

    def body(x_hbm, out_ref, gath, stageA, send_buf, stageB,
             copy_semA, copy_semB,
             zs, xs, ys, xds, yds,
             z_recv, xq_recv, yq_recv, xd_recv, yd_recv):
        my_x = lax.axis_index("x")
        my_y = lax.axis_index("y")
        my_z = lax.axis_index("z")
        zp = (my_x, my_y, 1 - my_z)
        xn = (1 - my_x, my_y, my_z)
        yn = (my_x, 1 - my_y, my_z)

        q = 2 * my_x + my_y
        qx = 2 * (1 - my_x) + my_y
        qy = 2 * my_x + (1 - my_y)
        qd = 2 * (1 - my_x) + (1 - my_y)

        barrier_sem = pltpu.get_barrier_semaphore()
        for nbr in (zp, xn, yn):
            pl.semaphore_signal(
                barrier_sem, inc=1, device_id=nbr,
                device_id_type=pl.DeviceIdType.MESH,
            )
        pl.semaphore_wait(barrier_sem, 3)

        peer_off = (1 - my_z) * n
        my_off = my_z * n

        def rows(r, s):
            return pl.ds(r * QR + s * SR, SR)

        def send(src_rows, dst_rows, send_sem, recv_sem, dev):
            r = pltpu.make_async_remote_copy(
                src_ref=gath.at[src_rows, :],
                dst_ref=gath.at[dst_rows, :],
                send_sem=send_sem,
                recv_sem=recv_sem,
                device_id=dev,
                device_id_type=pl.DeviceIdType.MESH,
            )
            r.start()
            return r

        def wait_recv(dst_rows, recv_sem):
            pltpu.make_async_remote_copy(
                src_ref=gath.at[dst_rows, :],
                dst_ref=gath.at[dst_rows, :],
                send_sem=zs.at[0],
                recv_sem=recv_sem,
                device_id=zp,
                device_id_type=pl.DeviceIdType.MESH,
            ).wait_recv()

        sends = []

        for s in range(S):
            slot = s % 2
            cp = pltpu.make_async_copy(
                x_hbm.at[0, rows(q, s), pl.ds(peer_off, n)],
                stageA.at[slot], copy_semA,
            )
            cp.start()
            cp.wait()
            send_buf[s, :, :] = stageA[slot, :, :].astype(jnp.bfloat16)
            r = pltpu.make_async_remote_copy(
                src_ref=send_buf.at[s],
                dst_ref=gath.at[rows(q, s), :],
                send_sem=zs.at[s],
                recv_sem=z_recv.at[s],
                device_id=zp,
                device_id_type=pl.DeviceIdType.MESH,
            )
            r.start()
            sends.append(r)

        for s in range(S):
            wait_recv(rows(q, s), z_recv.at[s])
            sends.append(send(rows(q, s), rows(q, s), xs.at[s],
                              xq_recv.at[s], xn))
            sends.append(send(rows(q, s), rows(q, s), ys.at[s],
                              yq_recv.at[s], yn))
            wait_recv(rows(qx, s), xq_recv.at[s])
            if s >= H:
                sends.append(send(rows(qx, s), rows(qx, s), yds.at[s - H],
                                  yd_recv.at[s - H], yn))
            wait_recv(rows(qy, s), yq_recv.at[s])
            if s < H:
                sends.append(send(rows(qy, s), rows(qy, s), xds.at[s],
                                  xd_recv.at[s], xn))

        rlist = [q, qx, qy, qd]

        def start_copyB(idx):
            cp = pltpu.make_async_copy(
                x_hbm.at[0, pl.ds(rlist[idx] * QR, QR), pl.ds(my_off, n)],
                stageB.at[idx % 2], copy_semB.at[idx % 2],
            )
            cp.start()
            return cp

        cps = [start_copyB(0)]
        for idx in range(4):
            if idx + 1 < 4:
                cps.append(start_copyB(idx + 1))
            if idx == 3:
                for s in range(H):
                    wait_recv(rows(qd, s), xd_recv.at[s])
                for s in range(H, S):
                    wait_recv(rows(qd, s), yd_recv.at[s - H])
            cps[idx].wait()
            r = rlist[idx]
            out_ref[pl.ds(r * QR, QR), :] = (
                gath[pl.ds(r * QR, QR), :]
                + stageB[idx % 2, :, :].astype(jnp.bfloat16)
            )

        for r in sends:
            r.wait_send()

    return pl.pallas_call(
        body,
        out_shape=jax.ShapeDtypeStruct((m, n), jnp.bfloat16),
        in_specs=[pl.BlockSpec(memory_space=pl.ANY)],
        out_specs=pl.BlockSpec(memory_space=pltpu.VMEM),
        scratch_shapes=[
            pltpu.VMEM((m, n), jnp.bfloat16),
            pltpu.VMEM((2, m // 4 // S, n), jnp.float32),
            pltpu.VMEM((S, m // 4 // S, n), jnp.bfloat16),
            pltpu.VMEM((2, m // 4, n), jnp.float32),
            pltpu.SemaphoreType.DMA,
            pltpu.SemaphoreType.DMA((2,)),
            pltpu.SemaphoreType.DMA((S,)),
            pltpu.SemaphoreType.DMA((S,)),
            pltpu.SemaphoreType.DMA((S,)),
            pltpu.SemaphoreType.DMA((H,)),
            pltpu.SemaphoreType.DMA((H,)),
            pltpu.SemaphoreType.DMA((S,)),
            pltpu.SemaphoreType.DMA((S,)),
            pltpu.SemaphoreType.DMA((S,)),
            pltpu.SemaphoreType.DMA((H,)),
            pltpu.SemaphoreType.DMA((H,)),
        ],
        compiler_params=pltpu.CompilerParams(
            collective_id=0,
            vmem_limit_bytes=60 * 1024 * 1024,
        ),
    )(x)


# device time: 133770 ns/iter; 1.4491x vs baseline; 1.0436x over previous
import jax
import jax.numpy as jnp
from jax import lax
from jax.experimental import pallas as pl
from jax.experimental.pallas import tpu as pltpu

S = 8
H = S // 2
NL = 6


def kernel(x):
    _, m, n2 = x.shape
    n = n2 // 2
    QR = m // 4
    SR = QR // S

    def body(x_hbm, out_ref, gath, stageA, send_buf, stageL,
             copy_semA, copy_semL,
             zs, xs, ys, xds, yds,
             z_recv, xq_recv, yq_recv, xd_recv, yd_recv):
        my_x = lax.axis_index("x")
        my_y = lax.axis_index("y")
        my_z = lax.axis_index("z")
        zp = (my_x, my_y, 1 - my_z)
        xn = (1 - my_x, my_y, my_z)
        yn = (my_x, 1 - my_y, my_z)

        q = 2 * my_x + my_y
        qx = 2 * (1 - my_x) + my_y
        qy = 2 * my_x + (1 - my_y)
        qd = 2 * (1 - my_x) + (1 - my_y)

        barrier_sem = pltpu.get_barrier_semaphore()
        for nbr in (zp, xn, yn):
            pl.semaphore_signal(
                barrier_sem, inc=1, device_id=nbr,
                device_id_type=pl.DeviceIdType.MESH,
            )
        pl.semaphore_wait(barrier_sem, 3)

        peer_off = (1 - my_z) * n
        my_off = my_z * n

        def rows(r, s):
            return pl.ds(r * QR + s * SR, SR)

        def send(src_rows, dst_rows, send_sem, recv_sem, dev):
            r = pltpu.make_async_remote_copy(
                src_ref=gath.at[src_rows, :],
                dst_ref=gath.at[dst_rows, :],
                send_sem=send_sem,
                recv_sem=recv_sem,
                device_id=dev,
                device_id_type=pl.DeviceIdType.MESH,
            )
            r.start()
            return r

        def wait_recv(dst_rows, recv_sem):
            pltpu.make_async_remote_copy(
                src_ref=gath.at[dst_rows, :],
                dst_ref=gath.at[dst_rows, :],
                send_sem=zs.at[0],
                recv_sem=recv_sem,
                device_id=zp,
                device_id_type=pl.DeviceIdType.MESH,
            ).wait_recv()

        local_order = []
        for s in range(S):
            local_order += [(q, s), (qx, s), (qy, s)]
        local_order += [(qd, s) for s in range(S)]

        local_cps = {}

        def start_local(k):
            if k >= len(local_order):
                return
            r, s = local_order[k]
            cp = pltpu.make_async_copy(
                x_hbm.at[0, rows(r, s), pl.ds(my_off, n)],
                stageL.at[k % NL], copy_semL.at[k % NL],
            )
            cp.start()
            local_cps[k] = cp

        def add_local(k):
            r, s = local_order[k]
            local_cps[k].wait()
            start_local(k + NL)
            out_ref[rows(r, s), :] = (
                gath[rows(r, s), :] + stageL[k % NL, :, :].astype(jnp.bfloat16)
            )

        for k in range(NL):
            start_local(k)

        sends = []

        cpsA = []
        for s in range(S):
            cpsA.append(pltpu.make_async_copy(
                x_hbm.at[0, rows(q, s), pl.ds(peer_off, n)],
                stageA.at[s % 2], copy_semA.at[s % 2],
            ))
        cpsA[0].start()
        for s in range(S):
            if s + 1 < S:
                cpsA[s + 1].start()
            cpsA[s].wait()
            send_buf[s, :, :] = stageA[s % 2, :, :].astype(jnp.bfloat16)
            r = pltpu.make_async_remote_copy(
                src_ref=send_buf.at[s],
                dst_ref=gath.at[rows(q, s), :],
                send_sem=zs.at[s],
                recv_sem=z_recv.at[s],
                device_id=zp,
                device_id_type=pl.DeviceIdType.MESH,
            )
            r.start()
            sends.append(r)

        for s in range(S):
            wait_recv(rows(q, s), z_recv.at[s])
            sends.append(send(rows(q, s), rows(q, s), xs.at[s],
                              xq_recv.at[s], xn))
            sends.append(send(rows(q, s), rows(q, s), ys.at[s],
                              yq_recv.at[s], yn))
            wait_recv(rows(qx, s), xq_recv.at[s])
            if s >= H:
                sends.append(send(rows(qx, s), rows(qx, s), yds.at[s - H],
                                  yd_recv.at[s - H], yn))
            wait_recv(rows(qy, s), yq_recv.at[s])
            if s < H:
                sends.append(send(rows(qy, s), rows(qy, s), xds.at[s],
                                  xd_recv.at[s], xn))
            add_local(3 * s)
            add_local(3 * s + 1)
            add_local(3 * s + 2)

        for s in range(S):
            if s < H:
                wait_recv(rows(qd, s), xd_recv.at[s])
            else:
                wait_recv(rows(qd, s), yd_recv.at[s - H])
            add_local(3 * S + s)

        for r in sends:
            r.wait_send()

    return pl.pallas_call(
        body,
        out_shape=jax.ShapeDtypeStruct((m, n), jnp.bfloat16),
        in_specs=[pl.BlockSpec(memory_space=pl.ANY)],
        out_specs=pl.BlockSpec(memory_space=pltpu.VMEM),
        scratch_shapes=[
            pltpu.VMEM((m, n), jnp.bfloat16),
            pltpu.VMEM((2, m // 4 // S, n), jnp.float32),
            pltpu.VMEM((S, m // 4 // S, n), jnp.bfloat16),
            pltpu.VMEM((NL, m // 4 // S, n), jnp.float32),
            pltpu.SemaphoreType.DMA((2,)),
            pltpu.SemaphoreType.DMA((NL,)),
            pltpu.SemaphoreType.DMA((S,)),
            pltpu.SemaphoreType.DMA((S,)),
            pltpu.SemaphoreType.DMA((S,)),
            pltpu.SemaphoreType.DMA((H,)),
            pltpu.SemaphoreType.DMA((H,)),
            pltpu.SemaphoreType.DMA((S,)),
            pltpu.SemaphoreType.DMA((S,)),
            pltpu.SemaphoreType.DMA((S,)),
            pltpu.SemaphoreType.DMA((H,)),
            pltpu.SemaphoreType.DMA((H,)),
        ],
        compiler_params=pltpu.CompilerParams(
            collective_id=0,
            vmem_limit_bytes=60 * 1024 * 1024,
        ),
    )(x)


# device time: 110039 ns/iter; 1.7616x vs baseline; 1.2157x over previous
import jax
import jax.numpy as jnp
from jax import lax
from jax.experimental import pallas as pl
from jax.experimental.pallas import tpu as pltpu

S = 8
H = S // 2
NL = 6


def kernel(x):
    _, m, n2 = x.shape
    n = n2 // 2
    QR = m // 4
    SR = QR // S

    def body(x_hbm, out_ref, gath, stageA, send_buf, stageL,
             copy_semA, copy_semL,
             zs, xs, ys, xds, yds,
             z_recv, xq_recv, yq_recv, xd_recv, yd_recv):
        my_x = lax.axis_index("x")
        my_y = lax.axis_index("y")
        my_z = lax.axis_index("z")
        zp = (my_x, my_y, 1 - my_z)
        xn = (1 - my_x, my_y, my_z)
        yn = (my_x, 1 - my_y, my_z)

        q = 2 * my_x + my_y
        qx = 2 * (1 - my_x) + my_y
        qy = 2 * my_x + (1 - my_y)
        qd = 2 * (1 - my_x) + (1 - my_y)

        barrier_sem = pltpu.get_barrier_semaphore()
        for nbr in (zp, xn, yn):
            pl.semaphore_signal(
                barrier_sem, inc=1, device_id=nbr,
                device_id_type=pl.DeviceIdType.MESH,
            )
        pl.semaphore_wait(barrier_sem, 3)

        peer_off = (1 - my_z) * n
        my_off = my_z * n

        def rows(r, s):
            return pl.ds(r * QR + s * SR, SR)

        def send(src_rows, dst_rows, send_sem, recv_sem, dev):
            r = pltpu.make_async_remote_copy(
                src_ref=gath.at[src_rows, :],
                dst_ref=gath.at[dst_rows, :],
                send_sem=send_sem,
                recv_sem=recv_sem,
                device_id=dev,
                device_id_type=pl.DeviceIdType.MESH,
            )
            r.start()
            return r

        def wait_recv(dst_rows, recv_sem):
            pltpu.make_async_remote_copy(
                src_ref=gath.at[dst_rows, :],
                dst_ref=gath.at[dst_rows, :],
                send_sem=zs.at[0],
                recv_sem=recv_sem,
                device_id=zp,
                device_id_type=pl.DeviceIdType.MESH,
            ).wait_recv()

        local_order = []
        for s in range(S):
            local_order += [(q, s), (qx, s), (qy, s)]
        local_order += [(qd, s) for s in range(S)]

        local_cps = {}

        def start_local(k):
            if k >= len(local_order):
                return
            r, s = local_order[k]
            cp = pltpu.make_async_copy(
                x_hbm.at[0, rows(r, s), pl.ds(my_off, n)],
                stageL.at[k % NL], copy_semL.at[k % NL],
            )
            cp.start()
            local_cps[k] = cp

        def add_local(k):
            r, s = local_order[k]
            local_cps[k].wait()
            start_local(k + NL)
            out_ref[rows(r, s), :] = (
                gath[rows(r, s), :] + stageL[k % NL, :, :].astype(jnp.bfloat16)
            )

        for k in range(NL):
            start_local(k)

        sends = []

        cpsA = []
        for s in range(S):
            cpsA.append(pltpu.make_async_copy(
                x_hbm.at[0, rows(q, s), pl.ds(peer_off, n)],
                stageA.at[s % 2], copy_semA.at[s % 2],
            ))
        cpsA[0].start()
        for s in range(S):
            if s + 1 < S:
                cpsA[s + 1].start()
            cpsA[s].wait()
            send_buf[s, :, :] = stageA[s % 2, :, :].astype(jnp.bfloat16)
            r = pltpu.make_async_remote_copy(
                src_ref=send_buf.at[s],
                dst_ref=gath.at[rows(q, s), :],
                send_sem=zs.at[s],
                recv_sem=z_recv.at[s],
                device_id=zp,
                device_id_type=pl.DeviceIdType.MESH,
            )
            r.start()
            sends.append(r)

        def consume(t):
            wait_recv(rows(qx, t), xq_recv.at[t])
            if t >= H:
                sends.append(send(rows(qx, t), rows(qx, t), yds.at[t - H],
                                  yd_recv.at[t - H], yn))
            wait_recv(rows(qy, t), yq_recv.at[t])
            if t < H:
                sends.append(send(rows(qy, t), rows(qy, t), xds.at[t],
                                  xd_recv.at[t], xn))
            add_local(3 * t)
            add_local(3 * t + 1)
            add_local(3 * t + 2)

        LAG = 2
        for s in range(S):
            wait_recv(rows(q, s), z_recv.at[s])
            sends.append(send(rows(q, s), rows(q, s), xs.at[s],
                              xq_recv.at[s], xn))
            sends.append(send(rows(q, s), rows(q, s), ys.at[s],
                              yq_recv.at[s], yn))
            if s >= LAG:
                consume(s - LAG)
        for t in range(S - LAG, S):
            consume(t)

        for s in range(S):
            if s < H:
                wait_recv(rows(qd, s), xd_recv.at[s])
            else:
                wait_recv(rows(qd, s), yd_recv.at[s - H])
            add_local(3 * S + s)

        for r in sends:
            r.wait_send()

    return pl.pallas_call(
        body,
        out_shape=jax.ShapeDtypeStruct((m, n), jnp.bfloat16),
        in_specs=[pl.BlockSpec(memory_space=pl.ANY)],
        out_specs=pl.BlockSpec(memory_space=pltpu.VMEM),
        scratch_shapes=[
            pltpu.VMEM((m, n), jnp.bfloat16),
            pltpu.VMEM((2, m // 4 // S, n), jnp.float32),
            pltpu.VMEM((S, m // 4 // S, n), jnp.bfloat16),
            pltpu.VMEM((NL, m // 4 // S, n), jnp.float32),
            pltpu.SemaphoreType.DMA((2,)),
            pltpu.SemaphoreType.DMA((NL,)),
            pltpu.SemaphoreType.DMA((S,)),
            pltpu.SemaphoreType.DMA((S,)),
            pltpu.SemaphoreType.DMA((S,)),
            pltpu.SemaphoreType.DMA((H,)),
            pltpu.SemaphoreType.DMA((H,)),
            pltpu.SemaphoreType.DMA((S,)),
            pltpu.SemaphoreType.DMA((S,)),
            pltpu.SemaphoreType.DMA((S,)),
            pltpu.SemaphoreType.DMA((H,)),
            pltpu.SemaphoreType.DMA((H,)),
        ],
        compiler_params=pltpu.CompilerParams(
            collective_id=0,
            vmem_limit_bytes=60 * 1024 * 1024,
        ),
    )(x)


# device time: 95217 ns/iter; 2.0358x vs baseline; 1.1557x over previous
import jax
import jax.numpy as jnp
from jax import lax
from jax.experimental import pallas as pl
from jax.experimental.pallas import tpu as pltpu

S = 8
ZD = 4
XD = ZD // 2
NL = 6


def kernel(x):
    _, m, n2 = x.shape
    n = n2 // 2
    QR = m // 4
    SR = QR // S

    def body(x_hbm, out_ref, gath, stageA, send_buf, stageL,
             copy_semA, copy_semL,
             zs, xs, ys, xds, yds,
             z_recv, xq_recv, yq_recv, xd_recv, yd_recv):
        my_x = lax.axis_index("x")
        my_y = lax.axis_index("y")
        my_z = lax.axis_index("z")
        zp = (my_x, my_y, 1 - my_z)
        xn = (1 - my_x, my_y, my_z)
        yn = (my_x, 1 - my_y, my_z)

        q = 2 * my_x + my_y
        qx = 2 * (1 - my_x) + my_y
        qy = 2 * my_x + (1 - my_y)
        qd = 2 * (1 - my_x) + (1 - my_y)

        barrier_sem = pltpu.get_barrier_semaphore()
        for nbr in (zp, xn, yn):
            pl.semaphore_signal(
                barrier_sem, inc=1, device_id=nbr,
                device_id_type=pl.DeviceIdType.MESH,
            )
        pl.semaphore_wait(barrier_sem, 3)

        peer_off = (1 - my_z) * n
        my_off = my_z * n

        def rows(r, s):
            return pl.ds(r * QR + s * SR, SR)

        def send(src_rows, dst_rows, send_sem, recv_sem, dev):
            r = pltpu.make_async_remote_copy(
                src_ref=gath.at[src_rows, :],
                dst_ref=gath.at[dst_rows, :],
                send_sem=send_sem,
                recv_sem=recv_sem,
                device_id=dev,
                device_id_type=pl.DeviceIdType.MESH,
            )
            r.start()
            return r

        def wait_recv(dst_rows, recv_sem):
            pltpu.make_async_remote_copy(
                src_ref=gath.at[dst_rows, :],
                dst_ref=gath.at[dst_rows, :],
                send_sem=zs.at[0],
                recv_sem=recv_sem,
                device_id=zp,
                device_id_type=pl.DeviceIdType.MESH,
            ).wait_recv()

        local_order = []
        for s in range(S):
            local_order += [(q, s), (qx, s), (qy, s)]
        local_order += [(qd, s) for s in range(S)]

        local_cps = {}

        def start_local(k):
            if k >= len(local_order):
                return
            r, s = local_order[k]
            cp = pltpu.make_async_copy(
                x_hbm.at[0, rows(r, s), pl.ds(my_off, n)],
                stageL.at[k % NL], copy_semL.at[k % NL],
            )
            cp.start()
            local_cps[k] = cp

        def add_local(k):
            r, s = local_order[k]
            local_cps[k].wait()
            start_local(k + NL)
            out_ref[rows(r, s), :] = (
                gath[rows(r, s), :] + stageL[k % NL, :, :].astype(jnp.bfloat16)
            )

        for k in range(NL):
            start_local(k)

        sends = []

        z_plan = [(q, s) for s in range(S)] + [(qd, s) for s in range(ZD, S)]
        NZ = len(z_plan)

        cpsA = []
        for i, (r_, s_) in enumerate(z_plan):
            cpsA.append(pltpu.make_async_copy(
                x_hbm.at[0, rows(r_, s_), pl.ds(peer_off, n)],
                stageA.at[i % 4], copy_semA.at[i % 4],
            ))
        for i in range(3):
            cpsA[i].start()

        def prep_z(i):
            if i + 3 < NZ:
                cpsA[i + 3].start()
            cpsA[i].wait()
            r_, s_ = z_plan[i]
            send_buf[i, :, :] = stageA[i % 4, :, :].astype(jnp.bfloat16)
            r = pltpu.make_async_remote_copy(
                src_ref=send_buf.at[i],
                dst_ref=gath.at[rows(r_, s_), :],
                send_sem=zs.at[i],
                recv_sem=z_recv.at[i],
                device_id=zp,
                device_id_type=pl.DeviceIdType.MESH,
            )
            r.start()
            sends.append(r)

        def fwd_own(u):
            wait_recv(rows(q, u), z_recv.at[u])
            sends.append(send(rows(q, u), rows(q, u), xs.at[u],
                              xq_recv.at[u], xn))
            sends.append(send(rows(q, u), rows(q, u), ys.at[u],
                              yq_recv.at[u], yn))

        def consume(t):
            wait_recv(rows(qx, t), xq_recv.at[t])
            if XD <= t < ZD:
                sends.append(send(rows(qx, t), rows(qx, t), yds.at[t - XD],
                                  yd_recv.at[t - XD], yn))
            wait_recv(rows(qy, t), yq_recv.at[t])
            if t < XD:
                sends.append(send(rows(qy, t), rows(qy, t), xds.at[t],
                                  xd_recv.at[t], xn))
            add_local(3 * t)
            add_local(3 * t + 1)
            add_local(3 * t + 2)

        ZLAG = 2
        LAG = 2
        for s in range(max(NZ, S + ZLAG + LAG)):
            if s < NZ:
                prep_z(s)
            if 0 <= s - ZLAG < S:
                fwd_own(s - ZLAG)
            if 0 <= s - ZLAG - LAG < S:
                consume(s - ZLAG - LAG)

        for s in range(S):
            if s < XD:
                wait_recv(rows(qd, s), xd_recv.at[s])
            elif s < ZD:
                wait_recv(rows(qd, s), yd_recv.at[s - XD])
            else:
                wait_recv(rows(qd, s), z_recv.at[S + s - ZD])
            add_local(3 * S + s)

        for r in sends:
            r.wait_send()

    return pl.pallas_call(
        body,
        out_shape=jax.ShapeDtypeStruct((m, n), jnp.bfloat16),
        in_specs=[pl.BlockSpec(memory_space=pl.ANY)],
        out_specs=pl.BlockSpec(memory_space=pltpu.VMEM),
        scratch_shapes=[
            pltpu.VMEM((m, n), jnp.bfloat16),
            pltpu.VMEM((4, m // 4 // S, n), jnp.float32),
            pltpu.VMEM((2 * S - ZD, m // 4 // S, n), jnp.bfloat16),
            pltpu.VMEM((NL, m // 4 // S, n), jnp.float32),
            pltpu.SemaphoreType.DMA((4,)),
            pltpu.SemaphoreType.DMA((NL,)),
            pltpu.SemaphoreType.DMA((2 * S - ZD,)),
            pltpu.SemaphoreType.DMA((S,)),
            pltpu.SemaphoreType.DMA((S,)),
            pltpu.SemaphoreType.DMA((XD,)),
            pltpu.SemaphoreType.DMA((ZD - XD,)),
            pltpu.SemaphoreType.DMA((2 * S - ZD,)),
            pltpu.SemaphoreType.DMA((S,)),
            pltpu.SemaphoreType.DMA((S,)),
            pltpu.SemaphoreType.DMA((XD,)),
            pltpu.SemaphoreType.DMA((ZD - XD,)),
        ],
        compiler_params=pltpu.CompilerParams(
            collective_id=0,
            vmem_limit_bytes=60 * 1024 * 1024,
        ),
    )(x)
